# baseline (speedup 1.0000x reference)
import functools

import jax
import jax.numpy as jnp
from jax import lax
from jax.experimental import pallas as pl
from jax.experimental.pallas import tpu as pltpu
import jax.experimental.pallas.tpu_sc as plsc

F = 26
D = 16
BATCH = 16384
DIN = F * D
DPAD = 512
NP = 4
K = 13
EPS = 1e-5
H1 = 512
H2 = 256
FIELD_SIZE = 40000

BM = 1024
NB = BATCH // BM

NW = 32
SPW = BATCH // NW
GS = 32
NG = SPW // GS


def _sc_gather(table, idx):
    mesh = plsc.VectorSubcoreMesh(core_axis_name="c", subcore_axis_name="s")

    @functools.partial(
        pl.kernel,
        out_type=(
            jax.ShapeDtypeStruct((NP * BATCH, 128), jnp.float32),
            jax.ShapeDtypeStruct((NW, 2, D), jnp.float32),
        ),
        mesh=mesh,
        compiler_params=pltpu.CompilerParams(use_tc_tiling_on_sc=False),
        scratch_types=[
            pltpu.VMEM((SPW, F), jnp.int32),
            pltpu.VMEM((GS * F, D), jnp.float32),
            pltpu.VMEM((GS * NP, 128), jnp.float32),
            pltpu.VMEM((2, D), jnp.float32),
            pltpu.SemaphoreType.DMA,
        ],
    )
    def gk(idx_hbm, tab_hbm, out_hbm, st_hbm, idx_v, bufa, bufb, stv, sem):
        wid = lax.axis_index("s") * 2 + lax.axis_index("c")
        b0 = wid * SPW
        pltpu.sync_copy(idx_hbm.at[pl.ds(b0, SPW)], idx_v)

        zero16 = jnp.zeros((D,), jnp.float32)
        for r in range(3 * GS, 4 * GS):
            for cc in range(2, 8):
                bufb[r, pl.ds(cc * D, D)] = zero16

        def group(g, carry):
            vsum, vsq = carry
            cps = []
            for ls in range(GS):
                cp = pltpu.async_copy(
                    tab_hbm.at[idx_v.at[g * GS + ls]],
                    bufa.at[pl.ds(ls * F, F)],
                    sem,
                )
                cps.append(cp)
            for cp in cps:
                cp.wait()

            def rep(ls, c2):
                s1, s2 = c2
                for f in range(F):
                    v = bufa[ls * F + f, :]
                    bufb[(f // 8) * GS + ls, pl.ds((f % 8) * D, D)] = v
                    s1 = s1 + v
                    s2 = s2 + v * v
                return (s1, s2)

            vsum, vsq = lax.fori_loop(0, GS, rep, (vsum, vsq))
            for p in range(NP):
                pltpu.sync_copy(
                    bufb.at[pl.ds(p * GS, GS)],
                    out_hbm.at[pl.ds(p * BATCH + b0 + g * GS, GS)],
                )
            return (vsum, vsq)

        vsum, vsq = lax.fori_loop(
            0, NG, group, (jnp.zeros((D,), jnp.float32), jnp.zeros((D,), jnp.float32))
        )
        stv[0, :] = vsum
        stv[1, :] = vsq
        pltpu.sync_copy(stv, st_hbm.at[wid])

    return gk(idx, table)


VOC = F * FIELD_SIZE
OROWS = VOC * D // 128
CHR = 208
NCH = OROWS // CHR


TCH = 3200


def _repack_body(t_ref, o_ref):
    x = t_ref[...]
    g = TCH // 8
    srow = lax.broadcasted_iota(jnp.int32, (D, D), 0)
    scol = lax.broadcasted_iota(jnp.int32, (D, D), 1)
    eye = (srow == scol).astype(jnp.float32)
    for j in range(8):
        o_ref[:, j * D : (j + 1) * D] = lax.dot_general(
            x[:, j * g : (j + 1) * g], eye,
            (((0,), (0,)), ((), ())),
            preferred_element_type=jnp.float32,
        )


def _repack_table(t16):
    return pl.pallas_call(
        _repack_body,
        grid=(VOC // TCH,),
        in_specs=[pl.BlockSpec((D, TCH), lambda i: (0, i))],
        out_specs=pl.BlockSpec((TCH // 8, 128), lambda i: (i, 0)),
        out_shape=jax.ShapeDtypeStruct((OROWS, 128), jnp.float32),
    )(t16)


def _ctrl_body(x0_ref, x1_ref, x2_ref, x3_ref, a_ref, w_ref, h_ref, s_ref):
    i = pl.program_id(0)
    xs = (x0_ref, x1_ref, x2_ref, x3_ref)
    h = jnp.zeros((BM, F), jnp.float32)
    for p in range(NP):
        xp = xs[p][...] * a_ref[p : p + 1, :]
        h = h + jnp.dot(xp, w_ref[p * 128 : (p + 1) * 128, :],
                        preferred_element_type=jnp.float32)
    h_ref[...] = h

    @pl.when(i == 0)
    def _():
        s_ref[...] = jnp.zeros_like(s_ref)

    s_ref[0:1, :] += jnp.sum(h, axis=0, keepdims=True)
    s_ref[1:2, :] += jnp.sum(h * h, axis=0, keepdims=True)


def _mask_body(x0_ref, x1_ref, x2_ref, x3_ref, h_ref, a_ref, c_ref, pq_ref,
               e_ref, w1_ref, y_ref, s_ref):
    i = pl.program_id(0)
    h = h_ref[...]
    hb = jnp.maximum(h * pq_ref[0:1, :] + pq_ref[1:2, :], 0.0)
    m = jnp.max(hb, axis=1, keepdims=True)
    e = jnp.exp(hb - m)
    w = e / jnp.sum(e, axis=1, keepdims=True)
    iota = lax.broadcasted_iota(jnp.int32, w.shape, 1)
    key = (lax.bitcast_convert_type(w, jnp.int32) & jnp.int32(~31)) | (31 - iota)
    sel = jnp.zeros(w.shape, dtype=jnp.bool_)
    for _ in range(K):
        mx = jnp.max(key, axis=1, keepdims=True)
        chosen = key == mx
        sel = sel | chosen
        key = jnp.where(chosen, jnp.int32(-1), key)
    wsel = jnp.where(sel, w, 0.0)
    maskw = wsel / jnp.sum(wsel, axis=1, keepdims=True)

    xs = (x0_ref, x1_ref, x2_ref, x3_ref)
    y = jnp.zeros((BM, H1), jnp.float32)
    for p in range(NP):
        mexp = jnp.dot(maskw, e_ref[:, p * 128 : (p + 1) * 128],
                       preferred_element_type=jnp.float32)
        xp = (xs[p][...] * a_ref[p : p + 1, :] + c_ref[p : p + 1, :]) * mexp
        y = y + jnp.dot(xp, w1_ref[p * 128 : (p + 1) * 128, :],
                        preferred_element_type=jnp.float32)
    y_ref[...] = y

    @pl.when(i == 0)
    def _():
        s_ref[...] = jnp.zeros_like(s_ref)

    s_ref[0:1, :] += jnp.sum(y, axis=0, keepdims=True)
    s_ref[1:2, :] += jnp.sum(y * y, axis=0, keepdims=True)


def _mlp_body(x_ref, pq_ref, w_ref, y_ref, s_ref):
    i = pl.program_id(0)
    z = jnp.maximum(x_ref[...] * pq_ref[0:1, :] + pq_ref[1:2, :], 0.0)
    y = jnp.dot(z, w_ref[...], preferred_element_type=jnp.float32)
    y_ref[...] = y

    @pl.when(i == 0)
    def _():
        s_ref[...] = jnp.zeros_like(s_ref)

    s_ref[0:1, :] += jnp.sum(y, axis=0, keepdims=True)
    s_ref[1:2, :] += jnp.sum(y * y, axis=0, keepdims=True)


def _loss_body(x_ref, pq_ref, wo_ref, bo_ref, t_ref, s_ref):
    i = pl.program_id(0)
    z = jnp.maximum(x_ref[...] * pq_ref[0:1, :] + pq_ref[1:2, :], 0.0)
    o = jnp.sum(z * wo_ref[...], axis=1, keepdims=True) + bo_ref[0, 0]
    r = 1.0 / (1.0 + jnp.exp(-o))
    rc = jnp.clip(r, 1e-7, 1.0 - 1e-7)
    t = t_ref[...]
    part = jnp.sum(t * jnp.log(rc) + (1.0 - t) * jnp.log(1.0 - rc))

    @pl.when(i == 0)
    def _():
        s_ref[...] = jnp.zeros_like(s_ref)

    s_ref[...] += part.reshape(1, 1)


def _pq(ssum, ssq, g, be, n):
    mu = ssum / n
    var = ssq / n - mu * mu
    p = g * lax.rsqrt(var + EPS)
    return jnp.stack([p, be - mu * p])


def _pad_rows(w):
    c = jnp.arange(DIN)
    f, d = c // D, c % D
    src = d * F + f
    dst = 128 * (f // 8) + D * (f % 8) + d
    out = jnp.zeros((DPAD, w.shape[1]), w.dtype)
    return out.at[dst].set(w[src])


def kernel(field, target, step, emb_table, g_bn, b_bn, W_ctrl, b_ctrl, g_ctrl,
           be_ctrl, W1, b1, g1, be1, W2, b2, g2, be2, Wo, bo):
    offsets = jnp.arange(F, dtype=jnp.int32) * FIELD_SIZE
    v = field + offsets[None, :]
    idx = (v // TCH) * TCH + (v % (TCH // 8)) * 8 + (v % TCH) // (TCH // 8)

    tlin = _repack_table(emb_table.T)
    xplanes, stats = _sc_gather(tlin.reshape(VOC, D), idx)

    Wc_p = _pad_rows(W_ctrl)
    W1_p = _pad_rows(W1)
    cpad = jnp.arange(DPAD)
    fpad = 8 * (cpad // 128) + (cpad % 128) // D
    fvalid = fpad < F
    expand = ((fpad[None, :] == jnp.arange(F)[:, None]) & fvalid[None, :]
              ).astype(jnp.float32)

    ssum = jnp.sum(stats, axis=0)
    n3 = float(BATCH * F)
    m_d = ssum[0] / n3
    v_d = ssum[1] / n3 - m_d * m_d
    inv_d = lax.rsqrt(v_d + EPS)
    a_d = g_bn * inv_d
    c_d = b_bn - g_bn * m_d * inv_d
    a_col = (jnp.tile(a_d, DPAD // D).reshape(NP, 128)
             * fvalid.reshape(NP, 128).astype(jnp.float32))
    c_col = (jnp.tile(c_d, DPAD // D).reshape(NP, 128)
             * fvalid.reshape(NP, 128).astype(jnp.float32))

    grid = (NB,)
    xspecs = [pl.BlockSpec((BM, 128), lambda i, p=p: (p * NB + i, 0))
              for p in range(NP)]
    xargs = (xplanes, xplanes, xplanes, xplanes)

    h, hstats = pl.pallas_call(
        _ctrl_body,
        grid=grid,
        in_specs=xspecs + [
            pl.BlockSpec((NP, 128), lambda i: (0, 0)),
            pl.BlockSpec((DPAD, F), lambda i: (0, 0)),
        ],
        out_specs=[
            pl.BlockSpec((BM, F), lambda i: (i, 0)),
            pl.BlockSpec((2, F), lambda i: (0, 0)),
        ],
        out_shape=[
            jax.ShapeDtypeStruct((BATCH, F), jnp.float32),
            jax.ShapeDtypeStruct((2, F), jnp.float32),
        ],
    )(*xargs, a_col, Wc_p)

    pq_h = _pq(hstats[0], hstats[1], g_ctrl, be_ctrl, float(BATCH))

    y1, s1 = pl.pallas_call(
        _mask_body,
        grid=grid,
        in_specs=xspecs + [
            pl.BlockSpec((BM, F), lambda i: (i, 0)),
            pl.BlockSpec((NP, 128), lambda i: (0, 0)),
            pl.BlockSpec((NP, 128), lambda i: (0, 0)),
            pl.BlockSpec((2, F), lambda i: (0, 0)),
            pl.BlockSpec((F, DPAD), lambda i: (0, 0)),
            pl.BlockSpec((DPAD, H1), lambda i: (0, 0)),
        ],
        out_specs=[
            pl.BlockSpec((BM, H1), lambda i: (i, 0)),
            pl.BlockSpec((2, H1), lambda i: (0, 0)),
        ],
        out_shape=[
            jax.ShapeDtypeStruct((BATCH, H1), jnp.float32),
            jax.ShapeDtypeStruct((2, H1), jnp.float32),
        ],
    )(*xargs, h, a_col, c_col, pq_h, expand, W1_p)

    pq1 = _pq(s1[0], s1[1], g1, be1, float(BATCH))

    y2, s2 = pl.pallas_call(
        _mlp_body,
        grid=grid,
        in_specs=[
            pl.BlockSpec((BM, H1), lambda i: (i, 0)),
            pl.BlockSpec((2, H1), lambda i: (0, 0)),
            pl.BlockSpec((H1, H2), lambda i: (0, 0)),
        ],
        out_specs=[
            pl.BlockSpec((BM, H2), lambda i: (i, 0)),
            pl.BlockSpec((2, H2), lambda i: (0, 0)),
        ],
        out_shape=[
            jax.ShapeDtypeStruct((BATCH, H2), jnp.float32),
            jax.ShapeDtypeStruct((2, H2), jnp.float32),
        ],
    )(y1, pq1, W2)

    pq2 = _pq(s2[0], s2[1], g2, be2, float(BATCH))

    acc = pl.pallas_call(
        _loss_body,
        grid=grid,
        in_specs=[
            pl.BlockSpec((BM, H2), lambda i: (i, 0)),
            pl.BlockSpec((2, H2), lambda i: (0, 0)),
            pl.BlockSpec((1, H2), lambda i: (0, 0)),
            pl.BlockSpec((1, 1), lambda i: (0, 0)),
            pl.BlockSpec((BM, 1), lambda i: (i, 0)),
        ],
        out_specs=pl.BlockSpec((1, 1), lambda i: (0, 0)),
        out_shape=jax.ShapeDtypeStruct((1, 1), jnp.float32),
    )(y2, pq2, Wo.T, bo.reshape(1, 1), target.reshape(BATCH, 1))

    return -acc[0, 0] / BATCH


# speedup vs baseline: 1.5964x; 1.5964x over previous
import functools

import jax
import jax.numpy as jnp
from jax import lax
from jax.experimental import pallas as pl
from jax.experimental.pallas import tpu as pltpu
import jax.experimental.pallas.tpu_sc as plsc

F = 26
D = 16
BATCH = 16384
DIN = F * D
DPAD = 512
NP = 4
K = 13
EPS = 1e-5
H1 = 512
H2 = 256
FIELD_SIZE = 40000

BM = 1024
NB = BATCH // BM

NW = 32
SPW = BATCH // NW
GS = 32
NG = SPW // GS


def _sc_gather(table, idx):
    mesh = plsc.VectorSubcoreMesh(core_axis_name="c", subcore_axis_name="s")

    @functools.partial(
        pl.kernel,
        out_type=(
            jax.ShapeDtypeStruct((NP * BATCH, 128), jnp.float32),
            jax.ShapeDtypeStruct((NW, 2, D), jnp.float32),
        ),
        mesh=mesh,
        compiler_params=pltpu.CompilerParams(use_tc_tiling_on_sc=False),
        scratch_types=[
            pltpu.VMEM((SPW, F), jnp.int32),
            pltpu.VMEM((GS * F, D), jnp.float32),
            pltpu.VMEM((GS * NP, 128), jnp.float32),
            pltpu.VMEM((2, D), jnp.float32),
            pltpu.SemaphoreType.DMA,
        ],
    )
    def gk(idx_hbm, tab_hbm, out_hbm, st_hbm, idx_v, bufa, bufb, stv, sem):
        wid = lax.axis_index("s") * 2 + lax.axis_index("c")
        b0 = wid * SPW
        pltpu.sync_copy(idx_hbm.at[pl.ds(b0, SPW)], idx_v)

        zero16 = jnp.zeros((D,), jnp.float32)
        for r in range(3 * GS, 4 * GS):
            for cc in range(2, 8):
                bufb[r, pl.ds(cc * D, D)] = zero16

        def group(g, carry):
            vsum, vsq = carry
            cps = []
            for ls in range(GS):
                cp = pltpu.async_copy(
                    tab_hbm.at[idx_v.at[g * GS + ls]],
                    bufa.at[pl.ds(ls * F, F)],
                    sem,
                )
                cps.append(cp)
            for cp in cps:
                cp.wait()

            def rep(ls, c2):
                s1, s2 = c2
                for f in range(F):
                    v = bufa[ls * F + f, :]
                    bufb[(f // 8) * GS + ls, pl.ds((f % 8) * D, D)] = v
                    s1 = s1 + v
                    s2 = s2 + v * v
                return (s1, s2)

            vsum, vsq = lax.fori_loop(0, GS, rep, (vsum, vsq))
            for p in range(NP):
                pltpu.sync_copy(
                    bufb.at[pl.ds(p * GS, GS)],
                    out_hbm.at[pl.ds(p * BATCH + b0 + g * GS, GS)],
                )
            return (vsum, vsq)

        vsum, vsq = lax.fori_loop(
            0, NG, group, (jnp.zeros((D,), jnp.float32), jnp.zeros((D,), jnp.float32))
        )
        stv[0, :] = vsum
        stv[1, :] = vsq
        pltpu.sync_copy(stv, st_hbm.at[wid])

    return gk(idx, table)


VOC = F * FIELD_SIZE
OROWS = VOC * D // 128
CHR = 208
NCH = OROWS // CHR


TCH = 3200


def _repack_body(t_ref, o_ref):
    x = t_ref[...]
    g = TCH // 8
    srow = lax.broadcasted_iota(jnp.int32, (D, 128), 0)
    scol = lax.broadcasted_iota(jnp.int32, (D, 128), 1)
    parts = []
    for j in range(8):
        ej = (scol == j * D + srow).astype(jnp.float32)
        parts.append(lax.dot_general(
            x[:, j * g : (j + 1) * g], ej,
            (((0,), (0,)), ((), ())),
            preferred_element_type=jnp.float32,
        ))
    while len(parts) > 1:
        parts = [a + b for a, b in zip(parts[::2], parts[1::2])]
    o_ref[...] = parts[0]


def _repack_table(t16):
    return pl.pallas_call(
        _repack_body,
        grid=(VOC // TCH,),
        in_specs=[pl.BlockSpec((D, TCH), lambda i: (0, i))],
        out_specs=pl.BlockSpec((TCH // 8, 128), lambda i: (i, 0)),
        out_shape=jax.ShapeDtypeStruct((OROWS, 128), jnp.float32),
    )(t16)


def _ctrl_body(x0_ref, x1_ref, x2_ref, x3_ref, a_ref, w_ref, h_ref, s_ref):
    i = pl.program_id(0)
    xs = (x0_ref, x1_ref, x2_ref, x3_ref)
    h = jnp.zeros((BM, F), jnp.float32)
    for p in range(NP):
        xp = xs[p][...] * a_ref[p : p + 1, :]
        h = h + jnp.dot(xp, w_ref[p * 128 : (p + 1) * 128, :],
                        preferred_element_type=jnp.float32)
    h_ref[...] = h

    @pl.when(i == 0)
    def _():
        s_ref[...] = jnp.zeros_like(s_ref)

    s_ref[0:1, :] += jnp.sum(h, axis=0, keepdims=True)
    s_ref[1:2, :] += jnp.sum(h * h, axis=0, keepdims=True)


def _mask_body(x0_ref, x1_ref, x2_ref, x3_ref, h_ref, a_ref, c_ref, pq_ref,
               e_ref, w1_ref, y_ref, s_ref):
    i = pl.program_id(0)
    h = h_ref[...]
    hb = jnp.maximum(h * pq_ref[0:1, :] + pq_ref[1:2, :], 0.0)
    m = jnp.max(hb, axis=1, keepdims=True)
    e = jnp.exp(hb - m)
    w = e / jnp.sum(e, axis=1, keepdims=True)
    iota = lax.broadcasted_iota(jnp.int32, w.shape, 1)
    key = (lax.bitcast_convert_type(w, jnp.int32) & jnp.int32(~31)) | (31 - iota)
    sel = jnp.zeros(w.shape, dtype=jnp.bool_)
    for _ in range(K):
        mx = jnp.max(key, axis=1, keepdims=True)
        chosen = key == mx
        sel = sel | chosen
        key = jnp.where(chosen, jnp.int32(-1), key)
    wsel = jnp.where(sel, w, 0.0)
    maskw = wsel / jnp.sum(wsel, axis=1, keepdims=True)

    xs = (x0_ref, x1_ref, x2_ref, x3_ref)
    y = jnp.zeros((BM, H1), jnp.float32)
    for p in range(NP):
        mexp = jnp.dot(maskw, e_ref[:, p * 128 : (p + 1) * 128],
                       preferred_element_type=jnp.float32)
        xp = (xs[p][...] * a_ref[p : p + 1, :] + c_ref[p : p + 1, :]) * mexp
        y = y + jnp.dot(xp, w1_ref[p * 128 : (p + 1) * 128, :],
                        preferred_element_type=jnp.float32)
    y_ref[...] = y

    @pl.when(i == 0)
    def _():
        s_ref[...] = jnp.zeros_like(s_ref)

    s_ref[0:1, :] += jnp.sum(y, axis=0, keepdims=True)
    s_ref[1:2, :] += jnp.sum(y * y, axis=0, keepdims=True)


def _mlp_body(x_ref, pq_ref, w_ref, y_ref, s_ref):
    i = pl.program_id(0)
    z = jnp.maximum(x_ref[...] * pq_ref[0:1, :] + pq_ref[1:2, :], 0.0)
    y = jnp.dot(z, w_ref[...], preferred_element_type=jnp.float32)
    y_ref[...] = y

    @pl.when(i == 0)
    def _():
        s_ref[...] = jnp.zeros_like(s_ref)

    s_ref[0:1, :] += jnp.sum(y, axis=0, keepdims=True)
    s_ref[1:2, :] += jnp.sum(y * y, axis=0, keepdims=True)


def _loss_body(x_ref, pq_ref, wo_ref, bo_ref, t_ref, s_ref):
    i = pl.program_id(0)
    z = jnp.maximum(x_ref[...] * pq_ref[0:1, :] + pq_ref[1:2, :], 0.0)
    o = jnp.sum(z * wo_ref[...], axis=1, keepdims=True) + bo_ref[0, 0]
    r = 1.0 / (1.0 + jnp.exp(-o))
    rc = jnp.clip(r, 1e-7, 1.0 - 1e-7)
    t = t_ref[...]
    part = jnp.sum(t * jnp.log(rc) + (1.0 - t) * jnp.log(1.0 - rc))

    @pl.when(i == 0)
    def _():
        s_ref[...] = jnp.zeros_like(s_ref)

    s_ref[...] += part.reshape(1, 1)


def _pq(ssum, ssq, g, be, n):
    mu = ssum / n
    var = ssq / n - mu * mu
    p = g * lax.rsqrt(var + EPS)
    return jnp.stack([p, be - mu * p])


def _pad_rows(w):
    c = jnp.arange(DIN)
    f, d = c // D, c % D
    src = d * F + f
    dst = 128 * (f // 8) + D * (f % 8) + d
    out = jnp.zeros((DPAD, w.shape[1]), w.dtype)
    return out.at[dst].set(w[src])


def kernel(field, target, step, emb_table, g_bn, b_bn, W_ctrl, b_ctrl, g_ctrl,
           be_ctrl, W1, b1, g1, be1, W2, b2, g2, be2, Wo, bo):
    offsets = jnp.arange(F, dtype=jnp.int32) * FIELD_SIZE
    v = field + offsets[None, :]
    idx = (v // TCH) * TCH + (v % (TCH // 8)) * 8 + (v % TCH) // (TCH // 8)

    tlin = _repack_table(emb_table.T)
    xplanes, stats = _sc_gather(tlin.reshape(VOC, D), idx)

    Wc_p = _pad_rows(W_ctrl)
    W1_p = _pad_rows(W1)
    cpad = jnp.arange(DPAD)
    fpad = 8 * (cpad // 128) + (cpad % 128) // D
    fvalid = fpad < F
    expand = ((fpad[None, :] == jnp.arange(F)[:, None]) & fvalid[None, :]
              ).astype(jnp.float32)

    ssum = jnp.sum(stats, axis=0)
    n3 = float(BATCH * F)
    m_d = ssum[0] / n3
    v_d = ssum[1] / n3 - m_d * m_d
    inv_d = lax.rsqrt(v_d + EPS)
    a_d = g_bn * inv_d
    c_d = b_bn - g_bn * m_d * inv_d
    a_col = (jnp.tile(a_d, DPAD // D).reshape(NP, 128)
             * fvalid.reshape(NP, 128).astype(jnp.float32))
    c_col = (jnp.tile(c_d, DPAD // D).reshape(NP, 128)
             * fvalid.reshape(NP, 128).astype(jnp.float32))

    grid = (NB,)
    xspecs = [pl.BlockSpec((BM, 128), lambda i, p=p: (p * NB + i, 0))
              for p in range(NP)]
    xargs = (xplanes, xplanes, xplanes, xplanes)

    h, hstats = pl.pallas_call(
        _ctrl_body,
        grid=grid,
        in_specs=xspecs + [
            pl.BlockSpec((NP, 128), lambda i: (0, 0)),
            pl.BlockSpec((DPAD, F), lambda i: (0, 0)),
        ],
        out_specs=[
            pl.BlockSpec((BM, F), lambda i: (i, 0)),
            pl.BlockSpec((2, F), lambda i: (0, 0)),
        ],
        out_shape=[
            jax.ShapeDtypeStruct((BATCH, F), jnp.float32),
            jax.ShapeDtypeStruct((2, F), jnp.float32),
        ],
    )(*xargs, a_col, Wc_p)

    pq_h = _pq(hstats[0], hstats[1], g_ctrl, be_ctrl, float(BATCH))

    y1, s1 = pl.pallas_call(
        _mask_body,
        grid=grid,
        in_specs=xspecs + [
            pl.BlockSpec((BM, F), lambda i: (i, 0)),
            pl.BlockSpec((NP, 128), lambda i: (0, 0)),
            pl.BlockSpec((NP, 128), lambda i: (0, 0)),
            pl.BlockSpec((2, F), lambda i: (0, 0)),
            pl.BlockSpec((F, DPAD), lambda i: (0, 0)),
            pl.BlockSpec((DPAD, H1), lambda i: (0, 0)),
        ],
        out_specs=[
            pl.BlockSpec((BM, H1), lambda i: (i, 0)),
            pl.BlockSpec((2, H1), lambda i: (0, 0)),
        ],
        out_shape=[
            jax.ShapeDtypeStruct((BATCH, H1), jnp.float32),
            jax.ShapeDtypeStruct((2, H1), jnp.float32),
        ],
    )(*xargs, h, a_col, c_col, pq_h, expand, W1_p)

    pq1 = _pq(s1[0], s1[1], g1, be1, float(BATCH))

    y2, s2 = pl.pallas_call(
        _mlp_body,
        grid=grid,
        in_specs=[
            pl.BlockSpec((BM, H1), lambda i: (i, 0)),
            pl.BlockSpec((2, H1), lambda i: (0, 0)),
            pl.BlockSpec((H1, H2), lambda i: (0, 0)),
        ],
        out_specs=[
            pl.BlockSpec((BM, H2), lambda i: (i, 0)),
            pl.BlockSpec((2, H2), lambda i: (0, 0)),
        ],
        out_shape=[
            jax.ShapeDtypeStruct((BATCH, H2), jnp.float32),
            jax.ShapeDtypeStruct((2, H2), jnp.float32),
        ],
    )(y1, pq1, W2)

    pq2 = _pq(s2[0], s2[1], g2, be2, float(BATCH))

    acc = pl.pallas_call(
        _loss_body,
        grid=grid,
        in_specs=[
            pl.BlockSpec((BM, H2), lambda i: (i, 0)),
            pl.BlockSpec((2, H2), lambda i: (0, 0)),
            pl.BlockSpec((1, H2), lambda i: (0, 0)),
            pl.BlockSpec((1, 1), lambda i: (0, 0)),
            pl.BlockSpec((BM, 1), lambda i: (i, 0)),
        ],
        out_specs=pl.BlockSpec((1, 1), lambda i: (0, 0)),
        out_shape=jax.ShapeDtypeStruct((1, 1), jnp.float32),
    )(y2, pq2, Wo.T, bo.reshape(1, 1), target.reshape(BATCH, 1))

    return -acc[0, 0] / BATCH


# speedup vs baseline: 2.2704x; 1.4222x over previous
import functools

import jax
import jax.numpy as jnp
from jax import lax
from jax.experimental import pallas as pl
from jax.experimental.pallas import tpu as pltpu
import jax.experimental.pallas.tpu_sc as plsc

F = 26
D = 16
BATCH = 16384
DIN = F * D
DPAD = 512
NP = 4
K = 13
EPS = 1e-5
H1 = 512
H2 = 256
FIELD_SIZE = 40000

BM = 1024
NB = BATCH // BM

NW = 32
SPW = BATCH // NW
GS = 32
NG = SPW // GS


def _sc_gather(table, idx):
    mesh = plsc.VectorSubcoreMesh(core_axis_name="c", subcore_axis_name="s")

    @functools.partial(
        pl.kernel,
        out_type=(
            jax.ShapeDtypeStruct((NP * BATCH, 128), jnp.float32),
            jax.ShapeDtypeStruct((NW, 2, D), jnp.float32),
        ),
        mesh=mesh,
        compiler_params=pltpu.CompilerParams(use_tc_tiling_on_sc=False),
        scratch_types=[
            pltpu.VMEM((SPW, F), jnp.int32),
            pltpu.VMEM((GS * F, D), jnp.float32),
            pltpu.VMEM((GS * NP, 128), jnp.float32),
            pltpu.VMEM((2, D), jnp.float32),
            pltpu.SemaphoreType.DMA,
        ],
    )
    def gk(idx_hbm, tab_hbm, out_hbm, st_hbm, idx_v, bufa, bufb, stv, sem):
        wid = lax.axis_index("s") * 2 + lax.axis_index("c")
        b0 = wid * SPW
        pltpu.sync_copy(idx_hbm.at[pl.ds(b0, SPW)], idx_v)

        zero16 = jnp.zeros((D,), jnp.float32)
        for r in range(3 * GS, 4 * GS):
            for cc in range(2, 8):
                bufb[r, pl.ds(cc * D, D)] = zero16

        def group(g, carry):
            vsum, vsq = carry
            cps = []
            for ls in range(GS):
                cp = pltpu.async_copy(
                    tab_hbm.at[idx_v.at[g * GS + ls]],
                    bufa.at[pl.ds(ls * F, F)],
                    sem,
                )
                cps.append(cp)
            for cp in cps:
                cp.wait()

            def rep(ls, c2):
                s1, s2 = c2
                for f in range(F):
                    v = bufa[ls * F + f, :]
                    bufb[(f // 8) * GS + ls, pl.ds((f % 8) * D, D)] = v
                    s1 = s1 + v
                    s2 = s2 + v * v
                return (s1, s2)

            vsum, vsq = lax.fori_loop(0, GS, rep, (vsum, vsq))
            for p in range(NP):
                pltpu.sync_copy(
                    bufb.at[pl.ds(p * GS, GS)],
                    out_hbm.at[pl.ds(p * BATCH + b0 + g * GS, GS)],
                )
            return (vsum, vsq)

        vsum, vsq = lax.fori_loop(
            0, NG, group, (jnp.zeros((D,), jnp.float32), jnp.zeros((D,), jnp.float32))
        )
        stv[0, :] = vsum
        stv[1, :] = vsq
        pltpu.sync_copy(stv, st_hbm.at[wid])

    return gk(idx, table)


VOC = F * FIELD_SIZE
OROWS = VOC * D // 128
CHR = 208
NCH = OROWS // CHR


TCH = 8320


def _repack_body(t_ref, o_ref):
    x = t_ref[...]
    g = TCH // 8
    x8 = jnp.concatenate([x[:, j * g : (j + 1) * g] for j in range(8)], axis=0)
    srow = lax.broadcasted_iota(jnp.int32, (128, 128), 0)
    scol = lax.broadcasted_iota(jnp.int32, (128, 128), 1)
    eye = (srow == scol).astype(jnp.float32)
    o_ref[...] = lax.dot_general(
        x8, eye, (((0,), (0,)), ((), ())),
        preferred_element_type=jnp.float32,
    )


def _repack_table(t16):
    return pl.pallas_call(
        _repack_body,
        grid=(VOC // TCH,),
        in_specs=[pl.BlockSpec((D, TCH), lambda i: (0, i))],
        out_specs=pl.BlockSpec((TCH // 8, 128), lambda i: (i, 0)),
        out_shape=jax.ShapeDtypeStruct((OROWS, 128), jnp.float32),
    )(t16)


def _ctrl_body(x0_ref, x1_ref, x2_ref, x3_ref, a_ref, w_ref, h_ref, s_ref):
    i = pl.program_id(0)
    xs = (x0_ref, x1_ref, x2_ref, x3_ref)
    h = jnp.zeros((BM, F), jnp.float32)
    for p in range(NP):
        xp = xs[p][...] * a_ref[p : p + 1, :]
        h = h + jnp.dot(xp, w_ref[p * 128 : (p + 1) * 128, :],
                        preferred_element_type=jnp.float32)
    h_ref[...] = h

    @pl.when(i == 0)
    def _():
        s_ref[...] = jnp.zeros_like(s_ref)

    s_ref[0:1, :] += jnp.sum(h, axis=0, keepdims=True)
    s_ref[1:2, :] += jnp.sum(h * h, axis=0, keepdims=True)


def _mask_body(x0_ref, x1_ref, x2_ref, x3_ref, h_ref, a_ref, c_ref, pq_ref,
               e_ref, w1_ref, y_ref, s_ref):
    i = pl.program_id(0)
    h = h_ref[...]
    hb = jnp.maximum(h * pq_ref[0:1, :] + pq_ref[1:2, :], 0.0)
    m = jnp.max(hb, axis=1, keepdims=True)
    e = jnp.exp(hb - m)
    w = e / jnp.sum(e, axis=1, keepdims=True)
    iota = lax.broadcasted_iota(jnp.int32, w.shape, 1)
    key = (lax.bitcast_convert_type(w, jnp.int32) & jnp.int32(~31)) | (31 - iota)
    sel = jnp.zeros(w.shape, dtype=jnp.bool_)
    for _ in range(K):
        mx = jnp.max(key, axis=1, keepdims=True)
        chosen = key == mx
        sel = sel | chosen
        key = jnp.where(chosen, jnp.int32(-1), key)
    wsel = jnp.where(sel, w, 0.0)
    maskw = wsel / jnp.sum(wsel, axis=1, keepdims=True)

    xs = (x0_ref, x1_ref, x2_ref, x3_ref)
    y = jnp.zeros((BM, H1), jnp.float32)
    for p in range(NP):
        mexp = jnp.dot(maskw, e_ref[:, p * 128 : (p + 1) * 128],
                       preferred_element_type=jnp.float32)
        xp = (xs[p][...] * a_ref[p : p + 1, :] + c_ref[p : p + 1, :]) * mexp
        y = y + jnp.dot(xp, w1_ref[p * 128 : (p + 1) * 128, :],
                        preferred_element_type=jnp.float32)
    y_ref[...] = y

    @pl.when(i == 0)
    def _():
        s_ref[...] = jnp.zeros_like(s_ref)

    s_ref[0:1, :] += jnp.sum(y, axis=0, keepdims=True)
    s_ref[1:2, :] += jnp.sum(y * y, axis=0, keepdims=True)


def _mlp_body(x_ref, pq_ref, w_ref, y_ref, s_ref):
    i = pl.program_id(0)
    z = jnp.maximum(x_ref[...] * pq_ref[0:1, :] + pq_ref[1:2, :], 0.0)
    y = jnp.dot(z, w_ref[...], preferred_element_type=jnp.float32)
    y_ref[...] = y

    @pl.when(i == 0)
    def _():
        s_ref[...] = jnp.zeros_like(s_ref)

    s_ref[0:1, :] += jnp.sum(y, axis=0, keepdims=True)
    s_ref[1:2, :] += jnp.sum(y * y, axis=0, keepdims=True)


def _loss_body(x_ref, pq_ref, wo_ref, bo_ref, t_ref, s_ref):
    i = pl.program_id(0)
    z = jnp.maximum(x_ref[...] * pq_ref[0:1, :] + pq_ref[1:2, :], 0.0)
    o = jnp.sum(z * wo_ref[...], axis=1, keepdims=True) + bo_ref[0, 0]
    r = 1.0 / (1.0 + jnp.exp(-o))
    rc = jnp.clip(r, 1e-7, 1.0 - 1e-7)
    t = t_ref[...]
    part = jnp.sum(t * jnp.log(rc) + (1.0 - t) * jnp.log(1.0 - rc))

    @pl.when(i == 0)
    def _():
        s_ref[...] = jnp.zeros_like(s_ref)

    s_ref[...] += part.reshape(1, 1)


def _pq(ssum, ssq, g, be, n):
    mu = ssum / n
    var = ssq / n - mu * mu
    p = g * lax.rsqrt(var + EPS)
    return jnp.stack([p, be - mu * p])


def _pad_rows(w):
    c = jnp.arange(DIN)
    f, d = c // D, c % D
    src = d * F + f
    dst = 128 * (f // 8) + D * (f % 8) + d
    out = jnp.zeros((DPAD, w.shape[1]), w.dtype)
    return out.at[dst].set(w[src])


def kernel(field, target, step, emb_table, g_bn, b_bn, W_ctrl, b_ctrl, g_ctrl,
           be_ctrl, W1, b1, g1, be1, W2, b2, g2, be2, Wo, bo):
    offsets = jnp.arange(F, dtype=jnp.int32) * FIELD_SIZE
    v = field + offsets[None, :]
    idx = (v // TCH) * TCH + (v % (TCH // 8)) * 8 + (v % TCH) // (TCH // 8)

    tlin = _repack_table(emb_table.T)
    xplanes, stats = _sc_gather(tlin.reshape(VOC, D), idx)

    Wc_p = _pad_rows(W_ctrl)
    W1_p = _pad_rows(W1)
    cpad = jnp.arange(DPAD)
    fpad = 8 * (cpad // 128) + (cpad % 128) // D
    fvalid = fpad < F
    expand = ((fpad[None, :] == jnp.arange(F)[:, None]) & fvalid[None, :]
              ).astype(jnp.float32)

    ssum = jnp.sum(stats, axis=0)
    n3 = float(BATCH * F)
    m_d = ssum[0] / n3
    v_d = ssum[1] / n3 - m_d * m_d
    inv_d = lax.rsqrt(v_d + EPS)
    a_d = g_bn * inv_d
    c_d = b_bn - g_bn * m_d * inv_d
    a_col = (jnp.tile(a_d, DPAD // D).reshape(NP, 128)
             * fvalid.reshape(NP, 128).astype(jnp.float32))
    c_col = (jnp.tile(c_d, DPAD // D).reshape(NP, 128)
             * fvalid.reshape(NP, 128).astype(jnp.float32))

    grid = (NB,)
    xspecs = [pl.BlockSpec((BM, 128), lambda i, p=p: (p * NB + i, 0))
              for p in range(NP)]
    xargs = (xplanes, xplanes, xplanes, xplanes)

    h, hstats = pl.pallas_call(
        _ctrl_body,
        grid=grid,
        in_specs=xspecs + [
            pl.BlockSpec((NP, 128), lambda i: (0, 0)),
            pl.BlockSpec((DPAD, F), lambda i: (0, 0)),
        ],
        out_specs=[
            pl.BlockSpec((BM, F), lambda i: (i, 0)),
            pl.BlockSpec((2, F), lambda i: (0, 0)),
        ],
        out_shape=[
            jax.ShapeDtypeStruct((BATCH, F), jnp.float32),
            jax.ShapeDtypeStruct((2, F), jnp.float32),
        ],
    )(*xargs, a_col, Wc_p)

    pq_h = _pq(hstats[0], hstats[1], g_ctrl, be_ctrl, float(BATCH))

    y1, s1 = pl.pallas_call(
        _mask_body,
        grid=grid,
        in_specs=xspecs + [
            pl.BlockSpec((BM, F), lambda i: (i, 0)),
            pl.BlockSpec((NP, 128), lambda i: (0, 0)),
            pl.BlockSpec((NP, 128), lambda i: (0, 0)),
            pl.BlockSpec((2, F), lambda i: (0, 0)),
            pl.BlockSpec((F, DPAD), lambda i: (0, 0)),
            pl.BlockSpec((DPAD, H1), lambda i: (0, 0)),
        ],
        out_specs=[
            pl.BlockSpec((BM, H1), lambda i: (i, 0)),
            pl.BlockSpec((2, H1), lambda i: (0, 0)),
        ],
        out_shape=[
            jax.ShapeDtypeStruct((BATCH, H1), jnp.float32),
            jax.ShapeDtypeStruct((2, H1), jnp.float32),
        ],
    )(*xargs, h, a_col, c_col, pq_h, expand, W1_p)

    pq1 = _pq(s1[0], s1[1], g1, be1, float(BATCH))

    y2, s2 = pl.pallas_call(
        _mlp_body,
        grid=grid,
        in_specs=[
            pl.BlockSpec((BM, H1), lambda i: (i, 0)),
            pl.BlockSpec((2, H1), lambda i: (0, 0)),
            pl.BlockSpec((H1, H2), lambda i: (0, 0)),
        ],
        out_specs=[
            pl.BlockSpec((BM, H2), lambda i: (i, 0)),
            pl.BlockSpec((2, H2), lambda i: (0, 0)),
        ],
        out_shape=[
            jax.ShapeDtypeStruct((BATCH, H2), jnp.float32),
            jax.ShapeDtypeStruct((2, H2), jnp.float32),
        ],
    )(y1, pq1, W2)

    pq2 = _pq(s2[0], s2[1], g2, be2, float(BATCH))

    acc = pl.pallas_call(
        _loss_body,
        grid=grid,
        in_specs=[
            pl.BlockSpec((BM, H2), lambda i: (i, 0)),
            pl.BlockSpec((2, H2), lambda i: (0, 0)),
            pl.BlockSpec((1, H2), lambda i: (0, 0)),
            pl.BlockSpec((1, 1), lambda i: (0, 0)),
            pl.BlockSpec((BM, 1), lambda i: (i, 0)),
        ],
        out_specs=pl.BlockSpec((1, 1), lambda i: (0, 0)),
        out_shape=jax.ShapeDtypeStruct((1, 1), jnp.float32),
    )(y2, pq2, Wo.T, bo.reshape(1, 1), target.reshape(BATCH, 1))

    return -acc[0, 0] / BATCH


# speedup vs baseline: 2.6207x; 1.1543x over previous
import functools

import jax
import jax.numpy as jnp
from jax import lax
from jax.experimental import pallas as pl
from jax.experimental.pallas import tpu as pltpu
import jax.experimental.pallas.tpu_sc as plsc

F = 26
D = 16
BATCH = 16384
DIN = F * D
DPAD = 512
NP = 4
K = 13
EPS = 1e-5
H1 = 512
H2 = 256
FIELD_SIZE = 40000

BM = 1024
NB = BATCH // BM

NW = 32
SPW = BATCH // NW
GS = 32
NG = SPW // GS


def _sc_gather(table, idx):
    mesh = plsc.VectorSubcoreMesh(core_axis_name="c", subcore_axis_name="s")

    @functools.partial(
        pl.kernel,
        out_type=(
            jax.ShapeDtypeStruct((NP * BATCH, 128), jnp.float32),
            jax.ShapeDtypeStruct((NW, 2, D), jnp.float32),
        ),
        mesh=mesh,
        compiler_params=pltpu.CompilerParams(use_tc_tiling_on_sc=False),
        scratch_types=[
            pltpu.VMEM((SPW, F), jnp.int32),
            pltpu.VMEM((GS * F, D), jnp.float32),
            pltpu.VMEM((GS * NP, 128), jnp.float32),
            pltpu.VMEM((2, D), jnp.float32),
            pltpu.SemaphoreType.DMA,
        ],
    )
    def gk(idx_hbm, tab_hbm, out_hbm, st_hbm, idx_v, bufa, bufb, stv, sem):
        wid = lax.axis_index("s") * 2 + lax.axis_index("c")
        b0 = wid * SPW
        pltpu.sync_copy(idx_hbm.at[pl.ds(b0, SPW)], idx_v)

        zero16 = jnp.zeros((D,), jnp.float32)
        for r in range(3 * GS, 4 * GS):
            for cc in range(2, 8):
                bufb[r, pl.ds(cc * D, D)] = zero16

        def group(g, carry):
            vsum, vsq = carry
            cps = []
            for ls in range(GS):
                cp = pltpu.async_copy(
                    tab_hbm.at[idx_v.at[g * GS + ls]],
                    bufa.at[pl.ds(ls * F, F)],
                    sem,
                )
                cps.append(cp)
            for cp in cps:
                cp.wait()

            def rep(ls, c2):
                s1, s2 = c2
                for f in range(F):
                    v = bufa[ls * F + f, :]
                    bufb[(f // 8) * GS + ls, pl.ds((f % 8) * D, D)] = v
                    s1 = s1 + v
                    s2 = s2 + v * v
                return (s1, s2)

            vsum, vsq = lax.fori_loop(0, GS, rep, (vsum, vsq))
            for p in range(NP):
                pltpu.sync_copy(
                    bufb.at[pl.ds(p * GS, GS)],
                    out_hbm.at[pl.ds(p * BATCH + b0 + g * GS, GS)],
                )
            return (vsum, vsq)

        vsum, vsq = lax.fori_loop(
            0, NG, group, (jnp.zeros((D,), jnp.float32), jnp.zeros((D,), jnp.float32))
        )
        stv[0, :] = vsum
        stv[1, :] = vsq
        pltpu.sync_copy(stv, st_hbm.at[wid])

    return gk(idx, table)


VOC = F * FIELD_SIZE
OROWS = VOC * D // 128
CHR = 208
NCH = OROWS // CHR


TCH = 41600


def _repack_body(t_ref, o_ref):
    x = t_ref[...]
    g = TCH // 8
    x8 = jnp.concatenate([x[:, j * g : (j + 1) * g] for j in range(8)], axis=0)
    srow = lax.broadcasted_iota(jnp.int32, (128, 128), 0)
    scol = lax.broadcasted_iota(jnp.int32, (128, 128), 1)
    eye = (srow == scol).astype(jnp.float32)
    o_ref[...] = lax.dot_general(
        x8, eye, (((0,), (0,)), ((), ())),
        preferred_element_type=jnp.float32,
    )


def _repack_table(t16):
    return pl.pallas_call(
        _repack_body,
        grid=(VOC // TCH,),
        in_specs=[pl.BlockSpec((D, TCH), lambda i: (0, i))],
        out_specs=pl.BlockSpec((TCH // 8, 128), lambda i: (i, 0)),
        out_shape=jax.ShapeDtypeStruct((OROWS, 128), jnp.float32),
    )(t16)


def _ctrl_body(x0_ref, x1_ref, x2_ref, x3_ref, a_ref, w_ref, h_ref, s_ref):
    i = pl.program_id(0)
    xs = (x0_ref, x1_ref, x2_ref, x3_ref)
    h = jnp.zeros((BM, F), jnp.float32)
    for p in range(NP):
        xp = xs[p][...] * a_ref[p : p + 1, :]
        h = h + jnp.dot(xp, w_ref[p * 128 : (p + 1) * 128, :],
                        preferred_element_type=jnp.float32)
    h_ref[...] = h

    @pl.when(i == 0)
    def _():
        s_ref[...] = jnp.zeros_like(s_ref)

    s_ref[0:1, :] += jnp.sum(h, axis=0, keepdims=True)
    s_ref[1:2, :] += jnp.sum(h * h, axis=0, keepdims=True)


def _mask_body(x0_ref, x1_ref, x2_ref, x3_ref, h_ref, a_ref, c_ref, pq_ref,
               e_ref, w1_ref, y_ref, s_ref):
    i = pl.program_id(0)
    h = h_ref[...]
    hb = jnp.maximum(h * pq_ref[0:1, :] + pq_ref[1:2, :], 0.0)
    m = jnp.max(hb, axis=1, keepdims=True)
    e = jnp.exp(hb - m)
    w = e / jnp.sum(e, axis=1, keepdims=True)
    iota = lax.broadcasted_iota(jnp.int32, w.shape, 1)
    key = (lax.bitcast_convert_type(w, jnp.int32) & jnp.int32(~31)) | (31 - iota)
    sel = jnp.zeros(w.shape, dtype=jnp.bool_)
    for _ in range(K):
        mx = jnp.max(key, axis=1, keepdims=True)
        chosen = key == mx
        sel = sel | chosen
        key = jnp.where(chosen, jnp.int32(-1), key)
    wsel = jnp.where(sel, w, 0.0)
    maskw = wsel / jnp.sum(wsel, axis=1, keepdims=True)

    xs = (x0_ref, x1_ref, x2_ref, x3_ref)
    y = jnp.zeros((BM, H1), jnp.float32)
    for p in range(NP):
        mexp = jnp.dot(maskw, e_ref[:, p * 128 : (p + 1) * 128],
                       preferred_element_type=jnp.float32)
        xp = (xs[p][...] * a_ref[p : p + 1, :] + c_ref[p : p + 1, :]) * mexp
        y = y + jnp.dot(xp, w1_ref[p * 128 : (p + 1) * 128, :],
                        preferred_element_type=jnp.float32)
    y_ref[...] = y

    @pl.when(i == 0)
    def _():
        s_ref[...] = jnp.zeros_like(s_ref)

    s_ref[0:1, :] += jnp.sum(y, axis=0, keepdims=True)
    s_ref[1:2, :] += jnp.sum(y * y, axis=0, keepdims=True)


def _mlp_body(x_ref, pq_ref, w_ref, y_ref, s_ref):
    i = pl.program_id(0)
    z = jnp.maximum(x_ref[...] * pq_ref[0:1, :] + pq_ref[1:2, :], 0.0)
    y = jnp.dot(z, w_ref[...], preferred_element_type=jnp.float32)
    y_ref[...] = y

    @pl.when(i == 0)
    def _():
        s_ref[...] = jnp.zeros_like(s_ref)

    s_ref[0:1, :] += jnp.sum(y, axis=0, keepdims=True)
    s_ref[1:2, :] += jnp.sum(y * y, axis=0, keepdims=True)


def _loss_body(x_ref, pq_ref, wo_ref, bo_ref, t_ref, s_ref):
    i = pl.program_id(0)
    z = jnp.maximum(x_ref[...] * pq_ref[0:1, :] + pq_ref[1:2, :], 0.0)
    o = jnp.sum(z * wo_ref[...], axis=1, keepdims=True) + bo_ref[0, 0]
    r = 1.0 / (1.0 + jnp.exp(-o))
    rc = jnp.clip(r, 1e-7, 1.0 - 1e-7)
    t = t_ref[...]
    part = jnp.sum(t * jnp.log(rc) + (1.0 - t) * jnp.log(1.0 - rc))

    @pl.when(i == 0)
    def _():
        s_ref[...] = jnp.zeros_like(s_ref)

    s_ref[...] += part.reshape(1, 1)


def _pq(ssum, ssq, g, be, n):
    mu = ssum / n
    var = ssq / n - mu * mu
    p = g * lax.rsqrt(var + EPS)
    return jnp.stack([p, be - mu * p])


def _pad_rows(w):
    c = jnp.arange(DIN)
    f, d = c // D, c % D
    src = d * F + f
    dst = 128 * (f // 8) + D * (f % 8) + d
    out = jnp.zeros((DPAD, w.shape[1]), w.dtype)
    return out.at[dst].set(w[src])


def kernel(field, target, step, emb_table, g_bn, b_bn, W_ctrl, b_ctrl, g_ctrl,
           be_ctrl, W1, b1, g1, be1, W2, b2, g2, be2, Wo, bo):
    offsets = jnp.arange(F, dtype=jnp.int32) * FIELD_SIZE
    v = field + offsets[None, :]
    idx = (v // TCH) * TCH + (v % (TCH // 8)) * 8 + (v % TCH) // (TCH // 8)

    tlin = _repack_table(emb_table.T)
    xplanes, stats = _sc_gather(tlin.reshape(VOC, D), idx)

    Wc_p = _pad_rows(W_ctrl)
    W1_p = _pad_rows(W1)
    cpad = jnp.arange(DPAD)
    fpad = 8 * (cpad // 128) + (cpad % 128) // D
    fvalid = fpad < F
    expand = ((fpad[None, :] == jnp.arange(F)[:, None]) & fvalid[None, :]
              ).astype(jnp.float32)

    ssum = jnp.sum(stats, axis=0)
    n3 = float(BATCH * F)
    m_d = ssum[0] / n3
    v_d = ssum[1] / n3 - m_d * m_d
    inv_d = lax.rsqrt(v_d + EPS)
    a_d = g_bn * inv_d
    c_d = b_bn - g_bn * m_d * inv_d
    a_col = (jnp.tile(a_d, DPAD // D).reshape(NP, 128)
             * fvalid.reshape(NP, 128).astype(jnp.float32))
    c_col = (jnp.tile(c_d, DPAD // D).reshape(NP, 128)
             * fvalid.reshape(NP, 128).astype(jnp.float32))

    grid = (NB,)
    xspecs = [pl.BlockSpec((BM, 128), lambda i, p=p: (p * NB + i, 0))
              for p in range(NP)]
    xargs = (xplanes, xplanes, xplanes, xplanes)

    h, hstats = pl.pallas_call(
        _ctrl_body,
        grid=grid,
        in_specs=xspecs + [
            pl.BlockSpec((NP, 128), lambda i: (0, 0)),
            pl.BlockSpec((DPAD, F), lambda i: (0, 0)),
        ],
        out_specs=[
            pl.BlockSpec((BM, F), lambda i: (i, 0)),
            pl.BlockSpec((2, F), lambda i: (0, 0)),
        ],
        out_shape=[
            jax.ShapeDtypeStruct((BATCH, F), jnp.float32),
            jax.ShapeDtypeStruct((2, F), jnp.float32),
        ],
    )(*xargs, a_col, Wc_p)

    pq_h = _pq(hstats[0], hstats[1], g_ctrl, be_ctrl, float(BATCH))

    y1, s1 = pl.pallas_call(
        _mask_body,
        grid=grid,
        in_specs=xspecs + [
            pl.BlockSpec((BM, F), lambda i: (i, 0)),
            pl.BlockSpec((NP, 128), lambda i: (0, 0)),
            pl.BlockSpec((NP, 128), lambda i: (0, 0)),
            pl.BlockSpec((2, F), lambda i: (0, 0)),
            pl.BlockSpec((F, DPAD), lambda i: (0, 0)),
            pl.BlockSpec((DPAD, H1), lambda i: (0, 0)),
        ],
        out_specs=[
            pl.BlockSpec((BM, H1), lambda i: (i, 0)),
            pl.BlockSpec((2, H1), lambda i: (0, 0)),
        ],
        out_shape=[
            jax.ShapeDtypeStruct((BATCH, H1), jnp.float32),
            jax.ShapeDtypeStruct((2, H1), jnp.float32),
        ],
    )(*xargs, h, a_col, c_col, pq_h, expand, W1_p)

    pq1 = _pq(s1[0], s1[1], g1, be1, float(BATCH))

    y2, s2 = pl.pallas_call(
        _mlp_body,
        grid=grid,
        in_specs=[
            pl.BlockSpec((BM, H1), lambda i: (i, 0)),
            pl.BlockSpec((2, H1), lambda i: (0, 0)),
            pl.BlockSpec((H1, H2), lambda i: (0, 0)),
        ],
        out_specs=[
            pl.BlockSpec((BM, H2), lambda i: (i, 0)),
            pl.BlockSpec((2, H2), lambda i: (0, 0)),
        ],
        out_shape=[
            jax.ShapeDtypeStruct((BATCH, H2), jnp.float32),
            jax.ShapeDtypeStruct((2, H2), jnp.float32),
        ],
    )(y1, pq1, W2)

    pq2 = _pq(s2[0], s2[1], g2, be2, float(BATCH))

    acc = pl.pallas_call(
        _loss_body,
        grid=grid,
        in_specs=[
            pl.BlockSpec((BM, H2), lambda i: (i, 0)),
            pl.BlockSpec((2, H2), lambda i: (0, 0)),
            pl.BlockSpec((1, H2), lambda i: (0, 0)),
            pl.BlockSpec((1, 1), lambda i: (0, 0)),
            pl.BlockSpec((BM, 1), lambda i: (i, 0)),
        ],
        out_specs=pl.BlockSpec((1, 1), lambda i: (0, 0)),
        out_shape=jax.ShapeDtypeStruct((1, 1), jnp.float32),
    )(y2, pq2, Wo.T, bo.reshape(1, 1), target.reshape(BATCH, 1))

    return -acc[0, 0] / BATCH


# speedup vs baseline: 3.1323x; 1.1952x over previous
import functools

import jax
import jax.numpy as jnp
from jax import lax
from jax.experimental import pallas as pl
from jax.experimental.pallas import tpu as pltpu
import jax.experimental.pallas.tpu_sc as plsc

F = 26
D = 16
BATCH = 16384
DIN = F * D
DPAD = 512
NP = 4
K = 13
EPS = 1e-5
H1 = 512
H2 = 256
FIELD_SIZE = 40000

BM = 1024
NB = BATCH // BM

NW = 32
SPW = BATCH // NW
GS = 32
NG = SPW // GS


def _sc_gather(table, idx):
    mesh = plsc.VectorSubcoreMesh(core_axis_name="c", subcore_axis_name="s")

    @functools.partial(
        pl.kernel,
        out_type=(
            jax.ShapeDtypeStruct((NP * BATCH, 128), jnp.float32),
            jax.ShapeDtypeStruct((NW, 2, D), jnp.float32),
        ),
        mesh=mesh,
        compiler_params=pltpu.CompilerParams(use_tc_tiling_on_sc=False),
        scratch_types=[
            pltpu.VMEM((SPW, F), jnp.int32),
            pltpu.VMEM((GS * F, D), jnp.float32),
            pltpu.VMEM((GS * NP, 128), jnp.float32),
            pltpu.VMEM((2, D), jnp.float32),
            pltpu.SemaphoreType.DMA,
        ],
    )
    def gk(idx_hbm, tab_hbm, out_hbm, st_hbm, idx_v, bufa, bufb, stv, sem):
        wid = lax.axis_index("s") * 2 + lax.axis_index("c")
        b0 = wid * SPW
        pltpu.sync_copy(idx_hbm.at[pl.ds(b0, SPW)], idx_v)

        zero16 = jnp.zeros((D,), jnp.float32)
        for r in range(3 * GS, 4 * GS):
            for cc in range(2, 8):
                bufb[r, pl.ds(cc * D, D)] = zero16

        def group(g, carry):
            vsum, vsq = carry
            cps = []
            for ls in range(GS):
                cp = pltpu.async_copy(
                    tab_hbm.at[idx_v.at[g * GS + ls]],
                    bufa.at[pl.ds(ls * F, F)],
                    sem,
                )
                cps.append(cp)
            for cp in cps:
                cp.wait()

            def rep(ls, c2):
                s1, s2 = c2
                for f in range(F):
                    v = bufa[ls * F + f, :]
                    bufb[(f // 8) * GS + ls, pl.ds((f % 8) * D, D)] = v
                    s1 = s1 + v
                    s2 = s2 + v * v
                return (s1, s2)

            vsum, vsq = lax.fori_loop(0, GS, rep, (vsum, vsq))
            for p in range(NP):
                pltpu.sync_copy(
                    bufb.at[pl.ds(p * GS, GS)],
                    out_hbm.at[pl.ds(p * BATCH + b0 + g * GS, GS)],
                )
            return (vsum, vsq)

        vsum, vsq = lax.fori_loop(
            0, NG, group, (jnp.zeros((D,), jnp.float32), jnp.zeros((D,), jnp.float32))
        )
        stv[0, :] = vsum
        stv[1, :] = vsq
        pltpu.sync_copy(stv, st_hbm.at[wid])

    return gk(idx, table)


VOC = F * FIELD_SIZE
OROWS = VOC * D // 128
CHR = 208
NCH = OROWS // CHR


TCH = 41600


def _repack_body(t_ref, o_ref):
    x = t_ref[...]
    g = TCH // 8
    x8 = jnp.concatenate([x[:, j * g : (j + 1) * g] for j in range(8)], axis=0)
    srow = lax.broadcasted_iota(jnp.int32, (128, 128), 0)
    scol = lax.broadcasted_iota(jnp.int32, (128, 128), 1)
    eye = (srow == scol).astype(jnp.float32)
    o_ref[...] = lax.dot_general(
        x8, eye, (((0,), (0,)), ((), ())),
        preferred_element_type=jnp.float32,
    )


def _repack_table(t16):
    return pl.pallas_call(
        _repack_body,
        grid=(VOC // TCH,),
        in_specs=[pl.BlockSpec((D, TCH), lambda i: (0, i))],
        out_specs=pl.BlockSpec((TCH // 8, 128), lambda i: (i, 0)),
        out_shape=jax.ShapeDtypeStruct((OROWS, 128), jnp.float32),
    )(t16)


def _ctrl_body(x0_ref, x1_ref, x2_ref, x3_ref, a_ref, w_ref, h_ref, s_ref):
    i = pl.program_id(0)
    xs = (x0_ref, x1_ref, x2_ref, x3_ref)
    ht = jnp.zeros((F, BM), jnp.float32)
    for p in range(NP):
        xp = xs[p][...] * a_ref[p : p + 1, :]
        ht = ht + lax.dot_general(
            w_ref[p * 128 : (p + 1) * 128, :], xp,
            (((0,), (1,)), ((), ())),
            preferred_element_type=jnp.float32)
    h_ref[...] = ht

    @pl.when(i == 0)
    def _():
        s_ref[...] = jnp.zeros_like(s_ref)

    s_ref[:, 0:1] += jnp.sum(ht, axis=1, keepdims=True)
    s_ref[:, 1:2] += jnp.sum(ht * ht, axis=1, keepdims=True)


def _mask_body(x0_ref, x1_ref, x2_ref, x3_ref, h_ref, a_ref, c_ref, pq_ref,
               e_ref, w1_ref, y_ref, s_ref):
    i = pl.program_id(0)
    ht = h_ref[...]
    hb = jnp.maximum(ht * pq_ref[:, 0:1] + pq_ref[:, 1:2], 0.0)
    m = jnp.max(hb, axis=0, keepdims=True)
    e = jnp.exp(hb - m)
    w = e / jnp.sum(e, axis=0, keepdims=True)
    iota = lax.broadcasted_iota(jnp.int32, w.shape, 0)
    key = (lax.bitcast_convert_type(w, jnp.int32) & jnp.int32(~31)) | (31 - iota)
    sel = jnp.zeros(w.shape, dtype=jnp.bool_)
    for _ in range(K):
        mx = jnp.max(key, axis=0, keepdims=True)
        chosen = key == mx
        sel = sel | chosen
        key = jnp.where(chosen, jnp.int32(-1), key)
    wsel = jnp.where(sel, w, 0.0)
    maskw = wsel / jnp.sum(wsel, axis=0, keepdims=True)

    xs = (x0_ref, x1_ref, x2_ref, x3_ref)
    y = jnp.zeros((BM, H1), jnp.float32)
    for p in range(NP):
        mexp = lax.dot_general(
            maskw, e_ref[:, p * 128 : (p + 1) * 128],
            (((0,), (0,)), ((), ())),
            preferred_element_type=jnp.float32)
        xp = (xs[p][...] * a_ref[p : p + 1, :] + c_ref[p : p + 1, :]) * mexp
        y = y + jnp.dot(xp, w1_ref[p * 128 : (p + 1) * 128, :],
                        preferred_element_type=jnp.float32)
    y_ref[...] = y

    @pl.when(i == 0)
    def _():
        s_ref[...] = jnp.zeros_like(s_ref)

    s_ref[0:1, :] += jnp.sum(y, axis=0, keepdims=True)
    s_ref[1:2, :] += jnp.sum(y * y, axis=0, keepdims=True)


def _mlp_body(x_ref, pq_ref, w_ref, y_ref, s_ref):
    i = pl.program_id(0)
    z = jnp.maximum(x_ref[...] * pq_ref[0:1, :] + pq_ref[1:2, :], 0.0)
    y = jnp.dot(z, w_ref[...], preferred_element_type=jnp.float32)
    y_ref[...] = y

    @pl.when(i == 0)
    def _():
        s_ref[...] = jnp.zeros_like(s_ref)

    s_ref[0:1, :] += jnp.sum(y, axis=0, keepdims=True)
    s_ref[1:2, :] += jnp.sum(y * y, axis=0, keepdims=True)


def _loss_body(x_ref, pq_ref, wo_ref, bo_ref, t_ref, s_ref):
    i = pl.program_id(0)
    z = jnp.maximum(x_ref[...] * pq_ref[0:1, :] + pq_ref[1:2, :], 0.0)
    o = jnp.sum(z * wo_ref[...], axis=1, keepdims=True) + bo_ref[0, 0]
    r = 1.0 / (1.0 + jnp.exp(-o))
    rc = jnp.clip(r, 1e-7, 1.0 - 1e-7)
    t = t_ref[...]
    part = jnp.sum(t * jnp.log(rc) + (1.0 - t) * jnp.log(1.0 - rc))

    @pl.when(i == 0)
    def _():
        s_ref[...] = jnp.zeros_like(s_ref)

    s_ref[...] += part.reshape(1, 1)


def _pq(ssum, ssq, g, be, n):
    mu = ssum / n
    var = ssq / n - mu * mu
    p = g * lax.rsqrt(var + EPS)
    return jnp.stack([p, be - mu * p])


def _pad_rows(w):
    c = jnp.arange(DIN)
    f, d = c // D, c % D
    src = d * F + f
    dst = 128 * (f // 8) + D * (f % 8) + d
    out = jnp.zeros((DPAD, w.shape[1]), w.dtype)
    return out.at[dst].set(w[src])


def kernel(field, target, step, emb_table, g_bn, b_bn, W_ctrl, b_ctrl, g_ctrl,
           be_ctrl, W1, b1, g1, be1, W2, b2, g2, be2, Wo, bo):
    offsets = jnp.arange(F, dtype=jnp.int32) * FIELD_SIZE
    v = field + offsets[None, :]
    idx = (v // TCH) * TCH + (v % (TCH // 8)) * 8 + (v % TCH) // (TCH // 8)

    tlin = _repack_table(emb_table.T)
    xplanes, stats = _sc_gather(tlin.reshape(VOC, D), idx)

    Wc_p = _pad_rows(W_ctrl)
    W1_p = _pad_rows(W1)
    cpad = jnp.arange(DPAD)
    fpad = 8 * (cpad // 128) + (cpad % 128) // D
    fvalid = fpad < F
    expand = ((fpad[None, :] == jnp.arange(F)[:, None]) & fvalid[None, :]
              ).astype(jnp.float32)

    ssum = jnp.sum(stats, axis=0)
    n3 = float(BATCH * F)
    m_d = ssum[0] / n3
    v_d = ssum[1] / n3 - m_d * m_d
    inv_d = lax.rsqrt(v_d + EPS)
    a_d = g_bn * inv_d
    c_d = b_bn - g_bn * m_d * inv_d
    a_col = (jnp.tile(a_d, DPAD // D).reshape(NP, 128)
             * fvalid.reshape(NP, 128).astype(jnp.float32))
    c_col = (jnp.tile(c_d, DPAD // D).reshape(NP, 128)
             * fvalid.reshape(NP, 128).astype(jnp.float32))

    grid = (NB,)
    xspecs = [pl.BlockSpec((BM, 128), lambda i, p=p: (p * NB + i, 0))
              for p in range(NP)]
    xargs = (xplanes, xplanes, xplanes, xplanes)

    h, hstats = pl.pallas_call(
        _ctrl_body,
        grid=grid,
        in_specs=xspecs + [
            pl.BlockSpec((NP, 128), lambda i: (0, 0)),
            pl.BlockSpec((DPAD, F), lambda i: (0, 0)),
        ],
        out_specs=[
            pl.BlockSpec((F, BM), lambda i: (0, i)),
            pl.BlockSpec((F, 2), lambda i: (0, 0)),
        ],
        out_shape=[
            jax.ShapeDtypeStruct((F, BATCH), jnp.float32),
            jax.ShapeDtypeStruct((F, 2), jnp.float32),
        ],
    )(*xargs, a_col, Wc_p)

    pq_h = _pq(hstats[:, 0], hstats[:, 1], g_ctrl, be_ctrl, float(BATCH)).T

    y1, s1 = pl.pallas_call(
        _mask_body,
        grid=grid,
        in_specs=xspecs + [
            pl.BlockSpec((F, BM), lambda i: (0, i)),
            pl.BlockSpec((NP, 128), lambda i: (0, 0)),
            pl.BlockSpec((NP, 128), lambda i: (0, 0)),
            pl.BlockSpec((F, 2), lambda i: (0, 0)),
            pl.BlockSpec((F, DPAD), lambda i: (0, 0)),
            pl.BlockSpec((DPAD, H1), lambda i: (0, 0)),
        ],
        out_specs=[
            pl.BlockSpec((BM, H1), lambda i: (i, 0)),
            pl.BlockSpec((2, H1), lambda i: (0, 0)),
        ],
        out_shape=[
            jax.ShapeDtypeStruct((BATCH, H1), jnp.float32),
            jax.ShapeDtypeStruct((2, H1), jnp.float32),
        ],
    )(*xargs, h, a_col, c_col, pq_h, expand, W1_p)

    pq1 = _pq(s1[0], s1[1], g1, be1, float(BATCH))

    y2, s2 = pl.pallas_call(
        _mlp_body,
        grid=grid,
        in_specs=[
            pl.BlockSpec((BM, H1), lambda i: (i, 0)),
            pl.BlockSpec((2, H1), lambda i: (0, 0)),
            pl.BlockSpec((H1, H2), lambda i: (0, 0)),
        ],
        out_specs=[
            pl.BlockSpec((BM, H2), lambda i: (i, 0)),
            pl.BlockSpec((2, H2), lambda i: (0, 0)),
        ],
        out_shape=[
            jax.ShapeDtypeStruct((BATCH, H2), jnp.float32),
            jax.ShapeDtypeStruct((2, H2), jnp.float32),
        ],
    )(y1, pq1, W2)

    pq2 = _pq(s2[0], s2[1], g2, be2, float(BATCH))

    acc = pl.pallas_call(
        _loss_body,
        grid=grid,
        in_specs=[
            pl.BlockSpec((BM, H2), lambda i: (i, 0)),
            pl.BlockSpec((2, H2), lambda i: (0, 0)),
            pl.BlockSpec((1, H2), lambda i: (0, 0)),
            pl.BlockSpec((1, 1), lambda i: (0, 0)),
            pl.BlockSpec((BM, 1), lambda i: (i, 0)),
        ],
        out_specs=pl.BlockSpec((1, 1), lambda i: (0, 0)),
        out_shape=jax.ShapeDtypeStruct((1, 1), jnp.float32),
    )(y2, pq2, Wo.T, bo.reshape(1, 1), target.reshape(BATCH, 1))

    return -acc[0, 0] / BATCH


# speedup vs baseline: 3.2038x; 1.0228x over previous
import functools

import jax
import jax.numpy as jnp
from jax import lax
from jax.experimental import pallas as pl
from jax.experimental.pallas import tpu as pltpu
import jax.experimental.pallas.tpu_sc as plsc

F = 26
D = 16
BATCH = 16384
DIN = F * D
DPAD = 512
NP = 4
K = 13
EPS = 1e-5
H1 = 512
H2 = 256
FIELD_SIZE = 40000

BM = 1024
NB = BATCH // BM

NW = 32
SPW = BATCH // NW
GS = 64
NG = SPW // GS


def _sc_gather(table, idx):
    mesh = plsc.VectorSubcoreMesh(core_axis_name="c", subcore_axis_name="s")

    @functools.partial(
        pl.kernel,
        out_type=(
            jax.ShapeDtypeStruct((NP * BATCH, 128), jnp.float32),
            jax.ShapeDtypeStruct((NW, 2, D), jnp.float32),
        ),
        mesh=mesh,
        compiler_params=pltpu.CompilerParams(use_tc_tiling_on_sc=False),
        scratch_types=[
            pltpu.VMEM((SPW, F), jnp.int32),
            pltpu.VMEM((GS * F, D), jnp.float32),
            pltpu.VMEM((GS * NP, 128), jnp.float32),
            pltpu.VMEM((2, D), jnp.float32),
            pltpu.SemaphoreType.DMA,
        ],
    )
    def gk(idx_hbm, tab_hbm, out_hbm, st_hbm, idx_v, bufa, bufb, stv, sem):
        wid = lax.axis_index("s") * 2 + lax.axis_index("c")
        b0 = wid * SPW
        pltpu.sync_copy(idx_hbm.at[pl.ds(b0, SPW)], idx_v)

        zero16 = jnp.zeros((D,), jnp.float32)
        for r in range(3 * GS, 4 * GS):
            for cc in range(2, 8):
                bufb[r, pl.ds(cc * D, D)] = zero16

        def group(g, carry):
            vsum, vsq = carry
            cps = []
            for ls in range(GS):
                cp = pltpu.async_copy(
                    tab_hbm.at[idx_v.at[g * GS + ls]],
                    bufa.at[pl.ds(ls * F, F)],
                    sem,
                )
                cps.append(cp)
            for cp in cps:
                cp.wait()

            def rep(ls, c2):
                s1, s2 = c2
                for f in range(F):
                    v = bufa[ls * F + f, :]
                    bufb[(f // 8) * GS + ls, pl.ds((f % 8) * D, D)] = v
                    s1 = s1 + v
                    s2 = s2 + v * v
                return (s1, s2)

            vsum, vsq = lax.fori_loop(0, GS, rep, (vsum, vsq))
            for p in range(NP):
                pltpu.sync_copy(
                    bufb.at[pl.ds(p * GS, GS)],
                    out_hbm.at[pl.ds(p * BATCH + b0 + g * GS, GS)],
                )
            return (vsum, vsq)

        vsum, vsq = lax.fori_loop(
            0, NG, group, (jnp.zeros((D,), jnp.float32), jnp.zeros((D,), jnp.float32))
        )
        stv[0, :] = vsum
        stv[1, :] = vsq
        pltpu.sync_copy(stv, st_hbm.at[wid])

    return gk(idx, table)


VOC = F * FIELD_SIZE
OROWS = VOC * D // 128
CHR = 208
NCH = OROWS // CHR


TCH = 41600


def _repack_body(t_ref, o_ref):
    x = t_ref[...]
    g = TCH // 8
    x8 = jnp.concatenate([x[:, j * g : (j + 1) * g] for j in range(8)], axis=0)
    srow = lax.broadcasted_iota(jnp.int32, (128, 128), 0)
    scol = lax.broadcasted_iota(jnp.int32, (128, 128), 1)
    eye = (srow == scol).astype(jnp.float32)
    o_ref[...] = lax.dot_general(
        x8, eye, (((0,), (0,)), ((), ())),
        preferred_element_type=jnp.float32,
    )


def _repack_table(t16):
    return pl.pallas_call(
        _repack_body,
        grid=(VOC // TCH,),
        in_specs=[pl.BlockSpec((D, TCH), lambda i: (0, i))],
        out_specs=pl.BlockSpec((TCH // 8, 128), lambda i: (i, 0)),
        out_shape=jax.ShapeDtypeStruct((OROWS, 128), jnp.float32),
    )(t16)


def _ctrl_body(x0_ref, x1_ref, x2_ref, x3_ref, a_ref, w_ref, h_ref, s_ref):
    i = pl.program_id(0)
    xs = (x0_ref, x1_ref, x2_ref, x3_ref)
    ht = jnp.zeros((F, BM), jnp.float32)
    for p in range(NP):
        xp = xs[p][...] * a_ref[p : p + 1, :]
        ht = ht + lax.dot_general(
            w_ref[p * 128 : (p + 1) * 128, :], xp,
            (((0,), (1,)), ((), ())),
            preferred_element_type=jnp.float32)
    h_ref[...] = ht

    @pl.when(i == 0)
    def _():
        s_ref[...] = jnp.zeros_like(s_ref)

    s_ref[:, 0:1] += jnp.sum(ht, axis=1, keepdims=True)
    s_ref[:, 1:2] += jnp.sum(ht * ht, axis=1, keepdims=True)


def _mask_body(x0_ref, x1_ref, x2_ref, x3_ref, h_ref, a_ref, c_ref, pq_ref,
               e_ref, w1_ref, y_ref, s_ref):
    i = pl.program_id(0)
    ht = h_ref[...]
    hb = jnp.maximum(ht * pq_ref[:, 0:1] + pq_ref[:, 1:2], 0.0)
    m = jnp.max(hb, axis=0, keepdims=True)
    e = jnp.exp(hb - m)
    w = e / jnp.sum(e, axis=0, keepdims=True)
    iota = lax.broadcasted_iota(jnp.int32, w.shape, 0)
    key = (lax.bitcast_convert_type(w, jnp.int32) & jnp.int32(~31)) | (31 - iota)
    sel = jnp.zeros(w.shape, dtype=jnp.bool_)
    for _ in range(K):
        mx = jnp.max(key, axis=0, keepdims=True)
        chosen = key == mx
        sel = sel | chosen
        key = jnp.where(chosen, jnp.int32(-1), key)
    wsel = jnp.where(sel, w, 0.0)
    maskw = wsel / jnp.sum(wsel, axis=0, keepdims=True)

    xs = (x0_ref, x1_ref, x2_ref, x3_ref)
    y = jnp.zeros((BM, H1), jnp.float32)
    for p in range(NP):
        mexp = lax.dot_general(
            maskw, e_ref[:, p * 128 : (p + 1) * 128],
            (((0,), (0,)), ((), ())),
            preferred_element_type=jnp.float32)
        xp = (xs[p][...] * a_ref[p : p + 1, :] + c_ref[p : p + 1, :]) * mexp
        y = y + jnp.dot(xp, w1_ref[p * 128 : (p + 1) * 128, :],
                        preferred_element_type=jnp.float32)
    y_ref[...] = y

    @pl.when(i == 0)
    def _():
        s_ref[...] = jnp.zeros_like(s_ref)

    s_ref[0:1, :] += jnp.sum(y, axis=0, keepdims=True)
    s_ref[1:2, :] += jnp.sum(y * y, axis=0, keepdims=True)


def _mlp_body(x_ref, pq_ref, w_ref, y_ref, s_ref):
    i = pl.program_id(0)
    z = jnp.maximum(x_ref[...] * pq_ref[0:1, :] + pq_ref[1:2, :], 0.0)
    y = jnp.dot(z, w_ref[...], preferred_element_type=jnp.float32)
    y_ref[...] = y

    @pl.when(i == 0)
    def _():
        s_ref[...] = jnp.zeros_like(s_ref)

    s_ref[0:1, :] += jnp.sum(y, axis=0, keepdims=True)
    s_ref[1:2, :] += jnp.sum(y * y, axis=0, keepdims=True)


def _loss_body(x_ref, pq_ref, wo_ref, bo_ref, t_ref, s_ref):
    i = pl.program_id(0)
    z = jnp.maximum(x_ref[...] * pq_ref[0:1, :] + pq_ref[1:2, :], 0.0)
    o = jnp.sum(z * wo_ref[...], axis=1, keepdims=True) + bo_ref[0, 0]
    r = 1.0 / (1.0 + jnp.exp(-o))
    rc = jnp.clip(r, 1e-7, 1.0 - 1e-7)
    t = t_ref[...]
    part = jnp.sum(t * jnp.log(rc) + (1.0 - t) * jnp.log(1.0 - rc))

    @pl.when(i == 0)
    def _():
        s_ref[...] = jnp.zeros_like(s_ref)

    s_ref[...] += part.reshape(1, 1)


def _pq(ssum, ssq, g, be, n):
    mu = ssum / n
    var = ssq / n - mu * mu
    p = g * lax.rsqrt(var + EPS)
    return jnp.stack([p, be - mu * p])


def _pad_rows(w):
    c = jnp.arange(DIN)
    f, d = c // D, c % D
    src = d * F + f
    dst = 128 * (f // 8) + D * (f % 8) + d
    out = jnp.zeros((DPAD, w.shape[1]), w.dtype)
    return out.at[dst].set(w[src])


def kernel(field, target, step, emb_table, g_bn, b_bn, W_ctrl, b_ctrl, g_ctrl,
           be_ctrl, W1, b1, g1, be1, W2, b2, g2, be2, Wo, bo):
    offsets = jnp.arange(F, dtype=jnp.int32) * FIELD_SIZE
    v = field + offsets[None, :]
    idx = (v // TCH) * TCH + (v % (TCH // 8)) * 8 + (v % TCH) // (TCH // 8)

    tlin = _repack_table(emb_table.T)
    xplanes, stats = _sc_gather(tlin.reshape(VOC, D), idx)

    Wc_p = _pad_rows(W_ctrl)
    W1_p = _pad_rows(W1)
    cpad = jnp.arange(DPAD)
    fpad = 8 * (cpad // 128) + (cpad % 128) // D
    fvalid = fpad < F
    expand = ((fpad[None, :] == jnp.arange(F)[:, None]) & fvalid[None, :]
              ).astype(jnp.float32)

    ssum = jnp.sum(stats, axis=0)
    n3 = float(BATCH * F)
    m_d = ssum[0] / n3
    v_d = ssum[1] / n3 - m_d * m_d
    inv_d = lax.rsqrt(v_d + EPS)
    a_d = g_bn * inv_d
    c_d = b_bn - g_bn * m_d * inv_d
    a_col = (jnp.tile(a_d, DPAD // D).reshape(NP, 128)
             * fvalid.reshape(NP, 128).astype(jnp.float32))
    c_col = (jnp.tile(c_d, DPAD // D).reshape(NP, 128)
             * fvalid.reshape(NP, 128).astype(jnp.float32))

    grid = (NB,)
    xspecs = [pl.BlockSpec((BM, 128), lambda i, p=p: (p * NB + i, 0))
              for p in range(NP)]
    xargs = (xplanes, xplanes, xplanes, xplanes)

    h, hstats = pl.pallas_call(
        _ctrl_body,
        grid=grid,
        in_specs=xspecs + [
            pl.BlockSpec((NP, 128), lambda i: (0, 0)),
            pl.BlockSpec((DPAD, F), lambda i: (0, 0)),
        ],
        out_specs=[
            pl.BlockSpec((F, BM), lambda i: (0, i)),
            pl.BlockSpec((F, 2), lambda i: (0, 0)),
        ],
        out_shape=[
            jax.ShapeDtypeStruct((F, BATCH), jnp.float32),
            jax.ShapeDtypeStruct((F, 2), jnp.float32),
        ],
    )(*xargs, a_col, Wc_p)

    pq_h = _pq(hstats[:, 0], hstats[:, 1], g_ctrl, be_ctrl, float(BATCH)).T

    y1, s1 = pl.pallas_call(
        _mask_body,
        grid=grid,
        in_specs=xspecs + [
            pl.BlockSpec((F, BM), lambda i: (0, i)),
            pl.BlockSpec((NP, 128), lambda i: (0, 0)),
            pl.BlockSpec((NP, 128), lambda i: (0, 0)),
            pl.BlockSpec((F, 2), lambda i: (0, 0)),
            pl.BlockSpec((F, DPAD), lambda i: (0, 0)),
            pl.BlockSpec((DPAD, H1), lambda i: (0, 0)),
        ],
        out_specs=[
            pl.BlockSpec((BM, H1), lambda i: (i, 0)),
            pl.BlockSpec((2, H1), lambda i: (0, 0)),
        ],
        out_shape=[
            jax.ShapeDtypeStruct((BATCH, H1), jnp.float32),
            jax.ShapeDtypeStruct((2, H1), jnp.float32),
        ],
    )(*xargs, h, a_col, c_col, pq_h, expand, W1_p)

    pq1 = _pq(s1[0], s1[1], g1, be1, float(BATCH))

    y2, s2 = pl.pallas_call(
        _mlp_body,
        grid=grid,
        in_specs=[
            pl.BlockSpec((BM, H1), lambda i: (i, 0)),
            pl.BlockSpec((2, H1), lambda i: (0, 0)),
            pl.BlockSpec((H1, H2), lambda i: (0, 0)),
        ],
        out_specs=[
            pl.BlockSpec((BM, H2), lambda i: (i, 0)),
            pl.BlockSpec((2, H2), lambda i: (0, 0)),
        ],
        out_shape=[
            jax.ShapeDtypeStruct((BATCH, H2), jnp.float32),
            jax.ShapeDtypeStruct((2, H2), jnp.float32),
        ],
    )(y1, pq1, W2)

    pq2 = _pq(s2[0], s2[1], g2, be2, float(BATCH))

    acc = pl.pallas_call(
        _loss_body,
        grid=grid,
        in_specs=[
            pl.BlockSpec((BM, H2), lambda i: (i, 0)),
            pl.BlockSpec((2, H2), lambda i: (0, 0)),
            pl.BlockSpec((1, H2), lambda i: (0, 0)),
            pl.BlockSpec((1, 1), lambda i: (0, 0)),
            pl.BlockSpec((BM, 1), lambda i: (i, 0)),
        ],
        out_specs=pl.BlockSpec((1, 1), lambda i: (0, 0)),
        out_shape=jax.ShapeDtypeStruct((1, 1), jnp.float32),
    )(y2, pq2, Wo.T, bo.reshape(1, 1), target.reshape(BATCH, 1))

    return -acc[0, 0] / BATCH
